# baseline (speedup 1.0000x reference)
import jax
import jax.numpy as jnp
from jax.experimental import pallas as pl
from jax.experimental.pallas import tpu as pltpu

_B = 16
_N = 131072
_D = 32
_BLK_N = 65536


def _weights_body(w0_ref, b0_ref, w1_ref, b1_ref, a_ref, c_ref):
    W0 = w0_ref[...]
    W1 = w1_ref[...]
    b0 = b0_ref[...]
    b1 = b1_ref[...]
    M = jnp.dot(W0.T, W1.T, preferred_element_type=jnp.float32)
    c = jnp.dot(b0, W1.T, preferred_element_type=jnp.float32) + b1
    M2 = jnp.dot(M, M, preferred_element_type=jnp.float32)
    M3 = jnp.dot(M2, M, preferred_element_type=jnp.float32)
    ctot = (jnp.dot(c, M2, preferred_element_type=jnp.float32)
            + jnp.dot(c, M, preferred_element_type=jnp.float32) + c)
    a_ref[...] = M3.T
    c_ref[...] = jnp.broadcast_to(ctot.T, (_D, 8))


def _stream_body(x_ref, a_ref, c_ref, o_ref):
    o_ref[0] = (jnp.dot(a_ref[...], x_ref[0],
                        preferred_element_type=jnp.float32)
                + c_ref[:, :1])


def kernel(x, W0, b0, W1, b1, g0, g1):
    del g0, g1
    A, C = pl.pallas_call(
        _weights_body,
        out_shape=(jax.ShapeDtypeStruct((_D, _D), jnp.float32),
                   jax.ShapeDtypeStruct((_D, 8), jnp.float32)),
    )(W0, b0.reshape(1, _D), W1, b1.reshape(1, _D))
    xt = jnp.transpose(x, (0, 2, 1))
    out = pl.pallas_call(
        _stream_body,
        grid=(_B, _N // _BLK_N),
        in_specs=[
            pl.BlockSpec((1, _D, _BLK_N), lambda b, i: (b, 0, i)),
            pl.BlockSpec((_D, _D), lambda b, i: (0, 0)),
            pl.BlockSpec((_D, 8), lambda b, i: (0, 0)),
        ],
        out_specs=pl.BlockSpec((1, _D, _BLK_N), lambda b, i: (b, 0, i)),
        out_shape=jax.ShapeDtypeStruct((_B, _D, _N), jnp.float32),
        compiler_params=pltpu.CompilerParams(
            dimension_semantics=("parallel", "parallel")),
    )(xt, A, C)
    return jnp.transpose(out, (0, 2, 1))


# speedup vs baseline: 1.0066x; 1.0066x over previous
import jax
import jax.numpy as jnp
from jax.experimental import pallas as pl
from jax.experimental.pallas import tpu as pltpu

_B = 16
_N = 131072
_D = 32
_BLK_N = 65536


def _body(x_ref, w0_ref, b0_ref, w1_ref, b1_ref, o_ref):
    W0 = w0_ref[...]
    W1 = w1_ref[...]
    b0 = b0_ref[...]
    b1 = b1_ref[...]
    M = jnp.dot(W0.T, W1.T, preferred_element_type=jnp.float32)
    c = jnp.dot(b0, W1.T, preferred_element_type=jnp.float32) + b1
    M2 = jnp.dot(M, M, preferred_element_type=jnp.float32)
    M3 = jnp.dot(M2, M, preferred_element_type=jnp.float32)
    ctot = (jnp.dot(c, M2, preferred_element_type=jnp.float32)
            + jnp.dot(c, M, preferred_element_type=jnp.float32) + c)
    o_ref[0] = (jnp.dot(M3.T, x_ref[0], preferred_element_type=jnp.float32)
                + ctot.T)


def kernel(x, W0, b0, W1, b1, g0, g1):
    del g0, g1
    xt = jnp.transpose(x, (0, 2, 1))
    out = pl.pallas_call(
        _body,
        grid=(_B, _N // _BLK_N),
        in_specs=[
            pl.BlockSpec((1, _D, _BLK_N), lambda b, i: (b, 0, i)),
            pl.BlockSpec((_D, _D), lambda b, i: (0, 0)),
            pl.BlockSpec((1, _D), lambda b, i: (0, 0)),
            pl.BlockSpec((_D, _D), lambda b, i: (0, 0)),
            pl.BlockSpec((1, _D), lambda b, i: (0, 0)),
        ],
        out_specs=pl.BlockSpec((1, _D, _BLK_N), lambda b, i: (b, 0, i)),
        out_shape=jax.ShapeDtypeStruct((_B, _D, _N), jnp.float32),
        compiler_params=pltpu.CompilerParams(
            dimension_semantics=("parallel", "parallel")),
    )(xt, W0, b0.reshape(1, _D), W1, b1.reshape(1, _D))
    return jnp.transpose(out, (0, 2, 1))
